# Initial kernel scaffold; baseline (speedup 1.0000x reference)
import jax
import jax.numpy as jnp
from jax.experimental import pallas as pl


def kernel(h, edge_follows, edge_likes, W0_f, b0_f, W0_l, b0_l, W1_f, b1_f, W1_l, b1_l):
    raise NotImplementedError("write your pallas kernel here")


# speedup vs baseline: 4.6769x; 4.6769x over previous
import functools

import jax
import jax.numpy as jnp
from jax import lax
from jax.experimental import pallas as pl
from jax.experimental.pallas import tpu as pltpu
from jax.experimental.pallas import tpu_sc as plsc

_NC = 2
_NS = 16
_K = 128


def _sc_mesh():
    return plsc.VectorSubcoreMesh(core_axis_name="c", subcore_axis_name="s")


def _make_deg_kernel(n, e):
    ept = e // _NS
    nfull = ept // _K
    rem = ept - nfull * _K
    rpt = (n // _NS) // 8 * 8
    tail = n - _NS * rpt

    @functools.partial(
        pl.kernel,
        mesh=_sc_mesh(),
        out_type=jax.ShapeDtypeStruct((2 * n, 16), jnp.float32),
        scratch_types=[
            pltpu.VMEM_SHARED((n, 16), jnp.float32),
            pltpu.VMEM((_K, 16), jnp.float32),
            pltpu.VMEM((_K, 16), jnp.float32),
            pltpu.VMEM((_K,), jnp.int32),
            pltpu.VMEM((_K,), jnp.int32),
            pltpu.VMEM((max(rem, 8),), jnp.int32),
            pltpu.VMEM((max(rem, 8),), jnp.int32),
            pltpu.VMEM((rpt, 16), jnp.float32),
        ],
    )
    def deg_kernel(src_hbm, dst_hbm, out_hbm, acc, ohs, ohd, sidx, didx,
                   sidx2, didx2, fbuf):
        c = lax.axis_index("c")
        s = lax.axis_index("s")
        lane = lax.iota(jnp.int32, 16)
        oh0 = jnp.where(lane == 0, 1.0, 0.0).astype(jnp.float32)
        oh1 = jnp.where(lane == 1, 1.0, 0.0).astype(jnp.float32)
        zero = jnp.zeros((16,), jnp.float32)

        def fill(i, _):
            ohs[i] = oh0
            ohd[i] = oh1
            return 0
        lax.fori_loop(0, _K, fill, 0)

        def zrow(i, _):
            fbuf[i] = zero
            return 0
        lax.fori_loop(0, rpt, zrow, 0)

        pltpu.sync_copy(fbuf, acc.at[pl.ds(s * rpt, rpt)])
        if tail:
            @pl.when(s == 0)
            def _():
                pltpu.sync_copy(fbuf.at[pl.ds(0, tail)],
                                acc.at[pl.ds(_NS * rpt, tail)])
        plsc.subcore_barrier()

        base = c * e + s * ept

        def body(i, _):
            off = base + i * _K
            pltpu.sync_copy(src_hbm.at[pl.ds(off, _K)], sidx)
            pltpu.sync_copy(dst_hbm.at[pl.ds(off, _K)], didx)
            pltpu.sync_copy(ohs, acc.at[sidx], add=True)
            pltpu.sync_copy(ohd, acc.at[didx], add=True)
            return 0
        lax.fori_loop(0, nfull, body, 0)

        if rem:
            off = base + nfull * _K
            pltpu.sync_copy(src_hbm.at[pl.ds(off, rem)], sidx2)
            pltpu.sync_copy(dst_hbm.at[pl.ds(off, rem)], didx2)
            pltpu.sync_copy(ohs.at[pl.ds(0, rem)], acc.at[sidx2], add=True)
            pltpu.sync_copy(ohd.at[pl.ds(0, rem)], acc.at[didx2], add=True)

        plsc.subcore_barrier()
        r0 = s * rpt
        pltpu.sync_copy(acc.at[pl.ds(r0, rpt)], fbuf)
        pltpu.sync_copy(fbuf, out_hbm.at[pl.ds(c * n + r0, rpt)])
        if tail:
            @pl.when(s == 0)
            def _():
                pltpu.sync_copy(acc.at[pl.ds(_NS * rpt, tail)],
                                fbuf.at[pl.ds(0, tail)])
                pltpu.sync_copy(fbuf.at[pl.ds(0, tail)],
                                out_hbm.at[pl.ds(c * n + _NS * rpt, tail)])

    return deg_kernel


def _make_edge_kernel(n, d, e):
    ept = e // _NS
    nfull = ept // _K
    rem = ept - nfull * _K
    rpt = (n // _NS) // 8 * 8
    tail = n - _NS * rpt
    nfl = 3
    fl = rpt // nfl
    assert fl * nfl == rpt and fl % 8 == 0

    @functools.partial(
        pl.kernel,
        mesh=_sc_mesh(),
        out_type=jax.ShapeDtypeStruct((2 * n, d), jnp.float32),
        scratch_types=[
            pltpu.VMEM_SHARED((n, d), jnp.float32),
            pltpu.VMEM((_K, d), jnp.float32),
            pltpu.VMEM((max(rem, 8), d), jnp.float32),
            pltpu.VMEM((_K,), jnp.int32),
            pltpu.VMEM((_K,), jnp.int32),
            pltpu.VMEM((max(rem, 8),), jnp.int32),
            pltpu.VMEM((max(rem, 8),), jnp.int32),
            pltpu.VMEM((fl, d), jnp.float32),
            pltpu.SemaphoreType.DMA,
        ],
    )
    def edge_kernel(table_hbm, src_hbm, dst_hbm, out_hbm, acc, msg, msg2,
                    sidx, didx, sidx2, didx2, fbuf, sem):
        c = lax.axis_index("c")
        s = lax.axis_index("s")
        zero = jnp.zeros((16,), jnp.float32)
        nlane = d // 16

        def zrow(i, _):
            def zcol(j, _):
                fbuf[i, pl.ds(j * 16, 16)] = zero
                return 0
            lax.fori_loop(0, nlane, zcol, 0)
            return 0
        lax.fori_loop(0, fl, zrow, 0)

        r0 = s * rpt
        for j in range(nfl):
            pltpu.sync_copy(fbuf, acc.at[pl.ds(r0 + j * fl, fl)])
        if tail:
            @pl.when(s == 0)
            def _():
                pltpu.sync_copy(fbuf.at[pl.ds(0, tail)],
                                acc.at[pl.ds(_NS * rpt, tail)])
        plsc.subcore_barrier()

        base = c * e + s * ept

        def body(i, _):
            off = base + i * _K
            pltpu.sync_copy(src_hbm.at[pl.ds(off, _K)], sidx)
            pltpu.sync_copy(dst_hbm.at[pl.ds(off, _K)], didx)
            pltpu.async_copy(table_hbm.at[sidx], msg, sem).wait()
            pltpu.sync_copy(msg, acc.at[didx], add=True)
            return 0
        lax.fori_loop(0, nfull, body, 0)

        if rem:
            off = base + nfull * _K
            pltpu.sync_copy(src_hbm.at[pl.ds(off, rem)], sidx2)
            pltpu.sync_copy(dst_hbm.at[pl.ds(off, rem)], didx2)
            pltpu.async_copy(table_hbm.at[sidx2], msg2, sem).wait()
            pltpu.sync_copy(msg2, acc.at[didx2], add=True)

        plsc.subcore_barrier()
        for j in range(nfl):
            pltpu.sync_copy(acc.at[pl.ds(r0 + j * fl, fl)], fbuf)
            pltpu.sync_copy(fbuf, out_hbm.at[pl.ds(c * n + r0 + j * fl, fl)])
        if tail:
            @pl.when(s == 0)
            def _():
                pltpu.sync_copy(acc.at[pl.ds(_NS * rpt, tail)],
                                fbuf.at[pl.ds(0, tail)])
                pltpu.sync_copy(fbuf.at[pl.ds(0, tail)],
                                out_hbm.at[pl.ds(c * n + _NS * rpt, tail)])

    return edge_kernel


_BR = 80


def _prep_body(h_ref, degf_ref, degl_ref, wf_ref, wl_ref,
               hw_ref, rdf_ref, rdl_ref):
    rdf = lax.rsqrt(jnp.maximum(degf_ref[...], 1.0))
    rdl = lax.rsqrt(jnp.maximum(degl_ref[...], 1.0))
    rdf_ref[...] = rdf
    rdl_ref[...] = rdl
    hb = h_ref[...]
    hw_ref[0] = jnp.dot(hb * rdf[:, 0:1], wf_ref[...],
                        preferred_element_type=jnp.float32)
    hw_ref[1] = jnp.dot(hb * rdl[:, 0:1], wl_ref[...],
                        preferred_element_type=jnp.float32)


def _mid_body(aggf_ref, aggl_ref, rdf_ref, rdl_ref, bf_ref, bl_ref,
              wf_ref, wl_ref, hw_ref):
    rdf = rdf_ref[...]
    rdl = rdl_ref[...]
    h0 = (aggf_ref[...] * rdf[:, 1:2] + bf_ref[0]
          + aggl_ref[...] * rdl[:, 1:2] + bl_ref[0])
    h0 = jnp.maximum(h0, 0.0)
    hw_ref[0] = jnp.dot(h0 * rdf[:, 0:1], wf_ref[...],
                        preferred_element_type=jnp.float32)
    hw_ref[1] = jnp.dot(h0 * rdl[:, 0:1], wl_ref[...],
                        preferred_element_type=jnp.float32)


def _post_body(aggf_ref, aggl_ref, rdf_ref, rdl_ref, bf_ref, bl_ref, out_ref):
    out_ref[...] = (aggf_ref[...] * rdf_ref[...][:, 1:2] + bf_ref[0]
                    + aggl_ref[...] * rdl_ref[...][:, 1:2] + bl_ref[0])


def _row_spec(d):
    return pl.BlockSpec((_BR, d), lambda i: (i, 0))


def _row_spec_off(d, off):
    return pl.BlockSpec((_BR, d), lambda i: (i + off, 0))


def _full_spec(shape):
    nd = len(shape)
    return pl.BlockSpec(shape, lambda i: (0,) * nd)


def kernel(h, edge_follows, edge_likes, W0_f, b0_f, W0_l, b0_l,
           W1_f, b1_f, W1_l, b1_l):
    n, d = h.shape
    e = edge_follows.shape[1]
    nb = n // _BR

    src_raw = jnp.concatenate([edge_follows[0], edge_likes[0]])
    dst_cat = jnp.concatenate([edge_follows[1], edge_likes[1]])
    src_gat = jnp.concatenate([edge_follows[0], edge_likes[0] + n])

    deg = _make_deg_kernel(n, e)(src_raw, dst_cat)
    edge = _make_edge_kernel(n, d, e)

    b0f = jnp.broadcast_to(b0_f, (8, d))
    b0l = jnp.broadcast_to(b0_l, (8, d))
    b1f = jnp.broadcast_to(b1_f, (8, d))
    b1l = jnp.broadcast_to(b1_l, (8, d))

    hw0, rdf, rdl = pl.pallas_call(
        _prep_body,
        grid=(nb,),
        in_specs=[
            _row_spec(d),
            _row_spec_off(16, 0),
            _row_spec_off(16, nb),
            _full_spec((d, d)),
            _full_spec((d, d)),
        ],
        out_specs=[
            pl.BlockSpec((2, _BR, d), lambda i: (0, i, 0)),
            _row_spec(16),
            _row_spec(16),
        ],
        out_shape=[
            jax.ShapeDtypeStruct((2, n, d), jnp.float32),
            jax.ShapeDtypeStruct((n, 16), jnp.float32),
            jax.ShapeDtypeStruct((n, 16), jnp.float32),
        ],
    )(h, deg, deg, W0_f, W0_l)

    agg0 = edge(hw0.reshape(2 * n, d), src_gat, dst_cat)

    hw1 = pl.pallas_call(
        _mid_body,
        grid=(nb,),
        in_specs=[
            _row_spec_off(d, 0),
            _row_spec_off(d, nb),
            _row_spec(16),
            _row_spec(16),
            _full_spec((8, d)),
            _full_spec((8, d)),
            _full_spec((d, d)),
            _full_spec((d, d)),
        ],
        out_specs=[pl.BlockSpec((2, _BR, d), lambda i: (0, i, 0))],
        out_shape=[jax.ShapeDtypeStruct((2, n, d), jnp.float32)],
    )(agg0, agg0, rdf, rdl, b0f, b0l, W1_f, W1_l)[0]

    agg1 = edge(hw1.reshape(2 * n, d), src_gat, dst_cat)

    out = pl.pallas_call(
        _post_body,
        grid=(nb,),
        in_specs=[
            _row_spec_off(d, 0),
            _row_spec_off(d, nb),
            _row_spec(16),
            _row_spec(16),
            _full_spec((8, d)),
            _full_spec((8, d)),
        ],
        out_specs=[_row_spec(d)],
        out_shape=[jax.ShapeDtypeStruct((n, d), jnp.float32)],
    )(agg1, agg1, rdf, rdl, b1f, b1l)[0]

    return out


# speedup vs baseline: 5.5926x; 1.1958x over previous
import functools

import jax
import jax.numpy as jnp
from jax import lax
from jax.experimental import pallas as pl
from jax.experimental.pallas import tpu as pltpu
from jax.experimental.pallas import tpu_sc as plsc

_NC = 2
_NS = 16
_K = 128


def _sc_mesh():
    return plsc.VectorSubcoreMesh(core_axis_name="c", subcore_axis_name="s")


def _make_deg_kernel(n, e):
    ept = e // _NS
    nfull = ept // _K
    rem = ept - nfull * _K
    rpt = (n // _NS) // 8 * 8
    tail = n - _NS * rpt

    @functools.partial(
        pl.kernel,
        mesh=_sc_mesh(),
        out_type=jax.ShapeDtypeStruct((2 * n, 16), jnp.float32),
        scratch_types=[
            pltpu.VMEM_SHARED((n, 16), jnp.float32),
            pltpu.VMEM((_K, 16), jnp.float32),
            pltpu.VMEM((_K, 16), jnp.float32),
            pltpu.VMEM((_K,), jnp.int32),
            pltpu.VMEM((_K,), jnp.int32),
            pltpu.VMEM((_K,), jnp.int32),
            pltpu.VMEM((_K,), jnp.int32),
            pltpu.VMEM((max(rem, 8),), jnp.int32),
            pltpu.VMEM((max(rem, 8),), jnp.int32),
            pltpu.VMEM((rpt, 16), jnp.float32),
            pltpu.SemaphoreType.DMA,
            pltpu.SemaphoreType.DMA,
            pltpu.SemaphoreType.DMA,
            pltpu.SemaphoreType.DMA,
        ],
    )
    def deg_kernel(src_hbm, dst_hbm, out_hbm, acc, ohs, ohd, sidx0, didx0,
                   sidx1, didx1, sidx2, didx2, fbuf, sm0a, sm0b, sm1a, sm1b):
        c = lax.axis_index("c")
        s = lax.axis_index("s")
        lane = lax.iota(jnp.int32, 16)
        oh0 = jnp.where(lane == 0, 1.0, 0.0).astype(jnp.float32)
        oh1 = jnp.where(lane == 1, 1.0, 0.0).astype(jnp.float32)
        zero = jnp.zeros((16,), jnp.float32)

        def fill(i, _):
            ohs[i] = oh0
            ohd[i] = oh1
            return 0
        lax.fori_loop(0, _K, fill, 0)

        def zrow(i, _):
            fbuf[i] = zero
            return 0
        lax.fori_loop(0, rpt, zrow, 0)

        pltpu.sync_copy(fbuf, acc.at[pl.ds(s * rpt, rpt)])
        if tail:
            @pl.when(s == 0)
            def _():
                pltpu.sync_copy(fbuf.at[pl.ds(0, tail)],
                                acc.at[pl.ds(_NS * rpt, tail)])
        plsc.subcore_barrier()

        base = c * e + s * ept
        bufs = ((sidx0, didx0, sm0a, sm0b), (sidx1, didx1, sm1a, sm1b))

        def dstage(i, buf):
            si, di, sa, sb = buf
            off = base + i * _K
            pltpu.sync_copy(src_hbm.at[pl.ds(off, _K)], si)
            pltpu.sync_copy(dst_hbm.at[pl.ds(off, _K)], di)
            pltpu.async_copy(ohs, acc.at[si], sa, add=True)
            pltpu.async_copy(ohd, acc.at[di], sb, add=True)

        def dwait(buf):
            si, di, sa, sb = buf
            pltpu.make_async_copy(ohs, acc.at[si], sa).wait()
            pltpu.make_async_copy(ohd, acc.at[di], sb).wait()

        assert nfull >= 2 and nfull % 2 == 0
        dstage(0, bufs[0])
        dstage(1, bufs[1])

        def body(i2, _):
            i = 2 * i2 + 2
            dwait(bufs[0])
            dstage(i, bufs[0])
            dwait(bufs[1])
            dstage(i + 1, bufs[1])
            return 0
        lax.fori_loop(0, (nfull - 2) // 2, body, 0)

        if rem:
            off = base + nfull * _K
            pltpu.sync_copy(src_hbm.at[pl.ds(off, rem)], sidx2)
            pltpu.sync_copy(dst_hbm.at[pl.ds(off, rem)], didx2)
            pltpu.sync_copy(ohs.at[pl.ds(0, rem)], acc.at[sidx2], add=True)
            pltpu.sync_copy(ohd.at[pl.ds(0, rem)], acc.at[didx2], add=True)
        dwait(bufs[0])
        dwait(bufs[1])

        plsc.subcore_barrier()
        r0 = s * rpt
        pltpu.sync_copy(acc.at[pl.ds(r0, rpt)], fbuf)
        pltpu.sync_copy(fbuf, out_hbm.at[pl.ds(c * n + r0, rpt)])
        if tail:
            @pl.when(s == 0)
            def _():
                pltpu.sync_copy(acc.at[pl.ds(_NS * rpt, tail)],
                                fbuf.at[pl.ds(0, tail)])
                pltpu.sync_copy(fbuf.at[pl.ds(0, tail)],
                                out_hbm.at[pl.ds(c * n + _NS * rpt, tail)])

    return deg_kernel


def _make_edge_kernel(n, d, e):
    ept = e // _NS
    nfull = ept // _K
    rem = ept - nfull * _K
    rpt = (n // _NS) // 8 * 8
    tail = n - _NS * rpt
    nfl = 13
    fl = rpt // nfl
    assert fl * nfl == rpt and fl % 8 == 0

    @functools.partial(
        pl.kernel,
        mesh=_sc_mesh(),
        out_type=jax.ShapeDtypeStruct((2 * n, d), jnp.float32),
        scratch_types=[
            pltpu.VMEM_SHARED((n, d), jnp.float32),
            pltpu.VMEM((_K, d), jnp.float32),
            pltpu.VMEM((_K, d), jnp.float32),
            pltpu.VMEM((max(rem, 8), d), jnp.float32),
            pltpu.VMEM((_K,), jnp.int32),
            pltpu.VMEM((_K,), jnp.int32),
            pltpu.VMEM((_K,), jnp.int32),
            pltpu.VMEM((_K,), jnp.int32),
            pltpu.VMEM((max(rem, 8),), jnp.int32),
            pltpu.VMEM((max(rem, 8),), jnp.int32),
            pltpu.VMEM((fl, d), jnp.float32),
            pltpu.SemaphoreType.DMA,
            pltpu.SemaphoreType.DMA,
            pltpu.SemaphoreType.DMA,
            pltpu.SemaphoreType.DMA,
            pltpu.SemaphoreType.DMA,
        ],
    )
    def edge_kernel(table_hbm, src_hbm, dst_hbm, out_hbm, acc, msg0, msg1,
                    msg2, sidx0, didx0, sidx1, didx1, sidx2, didx2, fbuf,
                    gsm0, gsm1, ssm0, ssm1, sem2):
        c = lax.axis_index("c")
        s = lax.axis_index("s")
        zero = jnp.zeros((16,), jnp.float32)
        nlane = d // 16

        def zrow(i, _):
            def zcol(j, _):
                fbuf[i, pl.ds(j * 16, 16)] = zero
                return 0
            lax.fori_loop(0, nlane, zcol, 0)
            return 0
        lax.fori_loop(0, fl, zrow, 0)

        r0 = s * rpt
        for j in range(nfl):
            pltpu.sync_copy(fbuf, acc.at[pl.ds(r0 + j * fl, fl)])
        if tail:
            @pl.when(s == 0)
            def _():
                pltpu.sync_copy(fbuf.at[pl.ds(0, tail)],
                                acc.at[pl.ds(_NS * rpt, tail)])
        plsc.subcore_barrier()

        base = c * e + s * ept
        bufs = ((sidx0, didx0, msg0, gsm0, ssm0),
                (sidx1, didx1, msg1, gsm1, ssm1))

        def stage(i, buf):
            si, di, mb, gs, ss = buf
            off = base + i * _K
            pltpu.sync_copy(src_hbm.at[pl.ds(off, _K)], si)
            pltpu.sync_copy(dst_hbm.at[pl.ds(off, _K)], di)
            pltpu.async_copy(table_hbm.at[si], mb, gs)

        def finish(buf):
            si, di, mb, gs, ss = buf
            pltpu.make_async_copy(table_hbm.at[si], mb, gs).wait()
            pltpu.async_copy(mb, acc.at[di], ss, add=True)

        def swait(buf):
            si, di, mb, gs, ss = buf
            pltpu.make_async_copy(mb, acc.at[di], ss).wait()

        assert nfull >= 2 and nfull % 2 == 0
        stage(0, bufs[0])
        finish(bufs[0])
        stage(1, bufs[1])

        def body(i2, _):
            i = 2 * i2 + 1
            finish(bufs[1])
            swait(bufs[0])
            stage(i + 1, bufs[0])
            finish(bufs[0])
            swait(bufs[1])
            stage(i + 2, bufs[1])
            return 0
        lax.fori_loop(0, (nfull - 2) // 2, body, 0)
        finish(bufs[1])

        if rem:
            off = base + nfull * _K
            pltpu.sync_copy(src_hbm.at[pl.ds(off, rem)], sidx2)
            pltpu.sync_copy(dst_hbm.at[pl.ds(off, rem)], didx2)
            pltpu.async_copy(table_hbm.at[sidx2], msg2, sem2)
            pltpu.make_async_copy(table_hbm.at[sidx2], msg2, sem2).wait()
            pltpu.sync_copy(msg2, acc.at[didx2], add=True)
        swait(bufs[0])
        swait(bufs[1])

        plsc.subcore_barrier()
        for j in range(nfl):
            pltpu.sync_copy(acc.at[pl.ds(r0 + j * fl, fl)], fbuf)
            pltpu.sync_copy(fbuf, out_hbm.at[pl.ds(c * n + r0 + j * fl, fl)])
        if tail:
            @pl.when(s == 0)
            def _():
                pltpu.sync_copy(acc.at[pl.ds(_NS * rpt, tail)],
                                fbuf.at[pl.ds(0, tail)])
                pltpu.sync_copy(fbuf.at[pl.ds(0, tail)],
                                out_hbm.at[pl.ds(c * n + _NS * rpt, tail)])

    return edge_kernel


_BR = 80


def _prep_body(h_ref, degf_ref, degl_ref, wf_ref, wl_ref,
               hw_ref, rdf_ref, rdl_ref):
    rdf = lax.rsqrt(jnp.maximum(degf_ref[...], 1.0))
    rdl = lax.rsqrt(jnp.maximum(degl_ref[...], 1.0))
    rdf_ref[...] = rdf
    rdl_ref[...] = rdl
    hb = h_ref[...]
    hw_ref[0] = jnp.dot(hb * rdf[:, 0:1], wf_ref[...],
                        preferred_element_type=jnp.float32)
    hw_ref[1] = jnp.dot(hb * rdl[:, 0:1], wl_ref[...],
                        preferred_element_type=jnp.float32)


def _mid_body(aggf_ref, aggl_ref, rdf_ref, rdl_ref, bf_ref, bl_ref,
              wf_ref, wl_ref, hw_ref):
    rdf = rdf_ref[...]
    rdl = rdl_ref[...]
    h0 = (aggf_ref[...] * rdf[:, 1:2] + bf_ref[0]
          + aggl_ref[...] * rdl[:, 1:2] + bl_ref[0])
    h0 = jnp.maximum(h0, 0.0)
    hw_ref[0] = jnp.dot(h0 * rdf[:, 0:1], wf_ref[...],
                        preferred_element_type=jnp.float32)
    hw_ref[1] = jnp.dot(h0 * rdl[:, 0:1], wl_ref[...],
                        preferred_element_type=jnp.float32)


def _post_body(aggf_ref, aggl_ref, rdf_ref, rdl_ref, bf_ref, bl_ref, out_ref):
    out_ref[...] = (aggf_ref[...] * rdf_ref[...][:, 1:2] + bf_ref[0]
                    + aggl_ref[...] * rdl_ref[...][:, 1:2] + bl_ref[0])


def _row_spec(d):
    return pl.BlockSpec((_BR, d), lambda i: (i, 0))


def _row_spec_off(d, off):
    return pl.BlockSpec((_BR, d), lambda i: (i + off, 0))


def _full_spec(shape):
    nd = len(shape)
    return pl.BlockSpec(shape, lambda i: (0,) * nd)


def kernel(h, edge_follows, edge_likes, W0_f, b0_f, W0_l, b0_l,
           W1_f, b1_f, W1_l, b1_l):
    n, d = h.shape
    e = edge_follows.shape[1]
    nb = n // _BR

    src_raw = jnp.concatenate([edge_follows[0], edge_likes[0]])
    dst_cat = jnp.concatenate([edge_follows[1], edge_likes[1]])
    src_gat = jnp.concatenate([edge_follows[0], edge_likes[0] + n])

    deg = _make_deg_kernel(n, e)(src_raw, dst_cat)
    edge = _make_edge_kernel(n, d, e)

    b0f = jnp.broadcast_to(b0_f, (8, d))
    b0l = jnp.broadcast_to(b0_l, (8, d))
    b1f = jnp.broadcast_to(b1_f, (8, d))
    b1l = jnp.broadcast_to(b1_l, (8, d))

    hw0, rdf, rdl = pl.pallas_call(
        _prep_body,
        grid=(nb,),
        in_specs=[
            _row_spec(d),
            _row_spec_off(16, 0),
            _row_spec_off(16, nb),
            _full_spec((d, d)),
            _full_spec((d, d)),
        ],
        out_specs=[
            pl.BlockSpec((2, _BR, d), lambda i: (0, i, 0)),
            _row_spec(16),
            _row_spec(16),
        ],
        out_shape=[
            jax.ShapeDtypeStruct((2, n, d), jnp.float32),
            jax.ShapeDtypeStruct((n, 16), jnp.float32),
            jax.ShapeDtypeStruct((n, 16), jnp.float32),
        ],
    )(h, deg, deg, W0_f, W0_l)

    agg0 = edge(hw0.reshape(2 * n, d), src_gat, dst_cat)

    hw1 = pl.pallas_call(
        _mid_body,
        grid=(nb,),
        in_specs=[
            _row_spec_off(d, 0),
            _row_spec_off(d, nb),
            _row_spec(16),
            _row_spec(16),
            _full_spec((8, d)),
            _full_spec((8, d)),
            _full_spec((d, d)),
            _full_spec((d, d)),
        ],
        out_specs=[pl.BlockSpec((2, _BR, d), lambda i: (0, i, 0))],
        out_shape=[jax.ShapeDtypeStruct((2, n, d), jnp.float32)],
    )(agg0, agg0, rdf, rdl, b0f, b0l, W1_f, W1_l)[0]

    agg1 = edge(hw1.reshape(2 * n, d), src_gat, dst_cat)

    out = pl.pallas_call(
        _post_body,
        grid=(nb,),
        in_specs=[
            _row_spec_off(d, 0),
            _row_spec_off(d, nb),
            _row_spec(16),
            _row_spec(16),
            _full_spec((8, d)),
            _full_spec((8, d)),
        ],
        out_specs=[_row_spec(d)],
        out_shape=[jax.ShapeDtypeStruct((n, d), jnp.float32)],
    )(agg1, agg1, rdf, rdl, b1f, b1l)[0]

    return out
